# baseline (speedup 1.0000x reference)
import jax
import jax.numpy as jnp
from jax.experimental import pallas as pl
from jax.experimental.pallas import tpu as pltpu

_N = 20000
_C = 20
_M = 20
_CB = 10
_CONF_T = 0.5
_IOU_T = 0.5
_NEG = -1e30


def _nms_dense_kernel(scores_ref, boxes_ref, out_b_ref, out_s_ref):
    y1 = boxes_ref[0:1, :]
    x1 = boxes_ref[1:2, :]
    y2 = boxes_ref[2:3, :]
    x2 = boxes_ref[3:4, :]
    a2 = jnp.maximum(y2 - y1, 0.0) * jnp.maximum(x2 - x1, 0.0)
    lane = jax.lax.broadcasted_iota(jnp.int32, (_CB, _N), 1)

    s0 = scores_ref[0]
    s0 = jnp.where(s0 >= _CONF_T, s0, _NEG)

    def step(i, s):
        m = jnp.max(s, axis=1, keepdims=True)
        idx = jnp.min(jnp.where(s == m, lane, _N), axis=1, keepdims=True)
        onehot = lane == idx

        def pick(coord):
            return jnp.sum(jnp.where(onehot, coord, 0.0), axis=1, keepdims=True)

        sy1 = pick(y1)
        sx1 = pick(x1)
        sy2 = pick(y2)
        sx2 = pick(x2)
        sa = jnp.maximum(sy2 - sy1, 0.0) * jnp.maximum(sx2 - sx1, 0.0)
        keep = m > (_NEG * 0.5)

        yy1 = jnp.maximum(sy1, y1)
        xx1 = jnp.maximum(sx1, x1)
        yy2 = jnp.minimum(sy2, y2)
        xx2 = jnp.minimum(sx2, x2)
        inter = jnp.maximum(yy2 - yy1, 0.0) * jnp.maximum(xx2 - xx1, 0.0)
        union = jnp.maximum(sa + a2 - inter, 1e-9)
        iou = inter / union
        suppress = jnp.logical_and(iou > _IOU_T, keep)
        s = jnp.where(jnp.logical_or(suppress, onehot), _NEG, s)

        kf = keep.astype(jnp.float32)
        out_b_ref[0, i] = jnp.concatenate([sy1, sx1, sy2, sx2], axis=1) * kf
        out_s_ref[0, i] = jnp.where(keep, m, 0.0)[:, 0]
        return s

    jax.lax.fori_loop(0, _M, step, s0, unroll=False)


def kernel(boxes, box_scores):
    g = _C // _CB
    scores_g = box_scores.T.reshape(g, _CB, _N)
    boxes_t = boxes.T
    out_b, out_s = pl.pallas_call(
        _nms_dense_kernel,
        grid=(g,),
        in_specs=[
            pl.BlockSpec((1, _CB, _N), lambda i: (i, 0, 0)),
            pl.BlockSpec((4, _N), lambda i: (0, 0)),
        ],
        out_specs=[
            pl.BlockSpec((1, _M, _CB, 4), lambda i: (i, 0, 0, 0)),
            pl.BlockSpec((1, _M, _CB), lambda i: (i, 0, 0)),
        ],
        out_shape=[
            jax.ShapeDtypeStruct((g, _M, _CB, 4), jnp.float32),
            jax.ShapeDtypeStruct((g, _M, _CB), jnp.float32),
        ],
        compiler_params=pltpu.CompilerParams(
            dimension_semantics=("parallel",),
        ),
    )(scores_g, boxes_t)
    box_array = out_b.transpose(0, 2, 1, 3).reshape(-1, 4)
    score_array = out_s.transpose(0, 2, 1).reshape(-1)
    class_array = jnp.repeat(jnp.arange(_C, dtype=jnp.int32), _M)
    return box_array, score_array, class_array


# speedup vs baseline: 1.5886x; 1.5886x over previous
import jax
import jax.numpy as jnp
from jax.experimental import pallas as pl

_N = 20000
_C = 20
_M = 20
_CONF_T = 0.5
_IOU_T = 0.5
_NEG = -1e30


def _nms_dense_kernel(scores_ref, boxes_ref, boxes_n4_ref, out_b_ref, out_s_ref):
    y1 = boxes_ref[0:1, :]
    x1 = boxes_ref[1:2, :]
    y2 = boxes_ref[2:3, :]
    x2 = boxes_ref[3:4, :]
    a2 = jnp.maximum(y2 - y1, 0.0) * jnp.maximum(x2 - x1, 0.0)
    lane = jax.lax.broadcasted_iota(jnp.int32, (_C, _N), 1)

    s0 = scores_ref[...]
    s0 = jnp.where(s0 >= _CONF_T, s0, _NEG)

    def step(i, s):
        m = jnp.max(s, axis=1, keepdims=True)
        idx = jnp.min(jnp.where(s == m, lane, _N - 1), axis=1, keepdims=True)
        onehot = lane == idx
        keep = m > (_NEG * 0.5)
        kf = keep.astype(jnp.float32)

        rows = [
            boxes_n4_ref[pl.ds(idx[c, 0], 1), :] for c in range(_C)
        ]
        sel = jnp.concatenate(rows, axis=0)
        selk = sel * kf
        sy1 = selk[:, 0:1]
        sx1 = selk[:, 1:2]
        sy2 = selk[:, 2:3]
        sx2 = selk[:, 3:4]
        sa = jnp.maximum(sy2 - sy1, 0.0) * jnp.maximum(sx2 - sx1, 0.0)

        yy1 = jnp.maximum(sy1, y1)
        xx1 = jnp.maximum(sx1, x1)
        yy2 = jnp.minimum(sy2, y2)
        xx2 = jnp.minimum(sx2, x2)
        inter = jnp.maximum(yy2 - yy1, 0.0) * jnp.maximum(xx2 - xx1, 0.0)
        union = jnp.maximum(sa + a2 - inter, 1e-9)
        iou = inter / union
        s = jnp.where(jnp.logical_or(iou > _IOU_T, onehot), _NEG, s)

        out_b_ref[i] = selk
        out_s_ref[i] = jnp.where(keep, m, 0.0)[:, 0]
        return s

    jax.lax.fori_loop(0, _M, step, s0, unroll=False)


def kernel(boxes, box_scores):
    scores_t = box_scores.T
    boxes_t = boxes.T
    out_b, out_s = pl.pallas_call(
        _nms_dense_kernel,
        out_shape=[
            jax.ShapeDtypeStruct((_M, _C, 4), jnp.float32),
            jax.ShapeDtypeStruct((_M, _C), jnp.float32),
        ],
    )(scores_t, boxes_t, boxes)
    box_array = out_b.transpose(1, 0, 2).reshape(-1, 4)
    score_array = out_s.T.reshape(-1)
    class_array = jnp.repeat(jnp.arange(_C, dtype=jnp.int32), _M)
    return box_array, score_array, class_array


# speedup vs baseline: 2.9233x; 1.8402x over previous
import jax
import jax.numpy as jnp
from jax.experimental import pallas as pl
from jax.experimental.pallas import tpu as pltpu

_N = 20000
_C = 20
_M = 20
_CONF_T = 0.5
_IOU_T = 0.5
_NEG = -1e30


def _nms_lazy_kernel(scores_ref, boxes_n4_ref, oy1_ref, ox1_ref, oy2_ref, ox2_ref, os_ref, s_ref):
    lane = jax.lax.broadcasted_iota(jnp.int32, (_C, _N), 1)
    slot = jax.lax.broadcasted_iota(jnp.int32, (_C, _M), 1)

    s0 = scores_ref[...]
    s_ref[...] = jnp.where(s0 >= _CONF_T, s0, _NEG)

    zcm = jnp.zeros((_C, _M), jnp.float32)
    init = (
        jnp.int32(1),
        jnp.zeros((_C, 1), jnp.int32),
        zcm, zcm, zcm, zcm,
        zcm,
    )

    def cond(carry):
        return carry[0] > 0

    def body(carry):
        _, cnt, ky1, kx1, ky2, kx2, ks = carry
        s = s_ref[...]
        m = jnp.max(s, axis=1, keepdims=True)
        idx = jnp.min(jnp.where(s == m, lane, _N - 1), axis=1, keepdims=True)
        s_ref[...] = jnp.where(lane == idx, _NEG, s)
        keep = m > (_NEG * 0.5)

        rows = [boxes_n4_ref[pl.ds(idx[c, 0], 1), :] for c in range(_C)]
        sel = jnp.concatenate(rows, axis=0)
        cy1 = sel[:, 0:1]
        cx1 = sel[:, 1:2]
        cy2 = sel[:, 2:3]
        cx2 = sel[:, 3:4]
        ca = jnp.maximum(cy2 - cy1, 0.0) * jnp.maximum(cx2 - cx1, 0.0)

        ka = jnp.maximum(ky2 - ky1, 0.0) * jnp.maximum(kx2 - kx1, 0.0)
        yy1 = jnp.maximum(ky1, cy1)
        xx1 = jnp.maximum(kx1, cx1)
        yy2 = jnp.minimum(ky2, cy2)
        xx2 = jnp.minimum(kx2, cx2)
        inter = jnp.maximum(yy2 - yy1, 0.0) * jnp.maximum(xx2 - xx1, 0.0)
        union = jnp.maximum(ka + ca - inter, 1e-9)
        iou = inter / union
        suppressed = jnp.any(iou > _IOU_T, axis=1, keepdims=True)

        accept = jnp.logical_and(keep, jnp.logical_not(suppressed))
        upd = jnp.logical_and(slot == cnt, accept)
        ky1 = jnp.where(upd, cy1, ky1)
        kx1 = jnp.where(upd, cx1, kx1)
        ky2 = jnp.where(upd, cy2, ky2)
        kx2 = jnp.where(upd, cx2, kx2)
        ks = jnp.where(upd, m, ks)
        cnt = cnt + accept.astype(jnp.int32)

        live_vec = jnp.logical_and(cnt < _M, keep).astype(jnp.int32)
        live = jnp.sum(live_vec, axis=0, keepdims=True)
        return (live[0, 0], cnt, ky1, kx1, ky2, kx2, ks)

    out = jax.lax.while_loop(cond, body, init)
    _, _, ky1, kx1, ky2, kx2, ks = out
    oy1_ref[...] = ky1
    ox1_ref[...] = kx1
    oy2_ref[...] = ky2
    ox2_ref[...] = kx2
    os_ref[...] = ks


def kernel(boxes, box_scores):
    scores_t = box_scores.T
    oy1, ox1, oy2, ox2, osc = pl.pallas_call(
        _nms_lazy_kernel,
        out_shape=[jax.ShapeDtypeStruct((_C, _M), jnp.float32)] * 5,
        scratch_shapes=[pltpu.VMEM((_C, _N), jnp.float32)],
    )(scores_t, boxes)
    box_array = jnp.stack([oy1, ox1, oy2, ox2], axis=-1).reshape(-1, 4)
    score_array = osc.reshape(-1)
    class_array = jnp.repeat(jnp.arange(_C, dtype=jnp.int32), _M)
    return box_array, score_array, class_array


# speedup vs baseline: 3.4375x; 1.1759x over previous
import jax
import jax.numpy as jnp
from jax.experimental import pallas as pl
from jax.experimental.pallas import tpu as pltpu

_N = 20000
_C = 20
_M = 20
_NB = 157
_NBP = 160
_NP = _NB * 128
_CONF_T = 0.5
_IOU_T = 0.5
_NEG = -1e30


def _nms_lazy_kernel(scores_ref, boxes_n4_ref, oy1_ref, ox1_ref, oy2_ref, ox2_ref, os_ref, s3_ref):
    lane_b = jax.lax.broadcasted_iota(jnp.int32, (_C, _NBP), 1)
    lane_i = jax.lax.broadcasted_iota(jnp.int32, (_C, 128), 1)
    slot = jax.lax.broadcasted_iota(jnp.int32, (_C, _M), 1)

    s0 = scores_ref[...]
    s3 = jnp.where(s0 >= _CONF_T, s0, _NEG)
    s3_ref[...] = s3
    b0 = jnp.concatenate(
        [jnp.max(s3, axis=2), jnp.full((_C, _NBP - _NB), _NEG, jnp.float32)],
        axis=1,
    )

    zcm = jnp.zeros((_C, _M), jnp.float32)
    init = (
        jnp.int32(1),
        jnp.zeros((_C, 1), jnp.int32),
        b0,
        zcm, zcm, zcm, zcm,
        zcm,
    )

    def cond(carry):
        return carry[0] > 0

    def body(carry):
        _, cnt, bmax, ky1, kx1, ky2, kx2, ks = carry
        m = jnp.max(bmax, axis=1, keepdims=True)
        j = jnp.min(jnp.where(bmax == m, lane_b, _NB - 1), axis=1, keepdims=True)
        keep = m > (_NEG * 0.5)

        blocks = jnp.concatenate(
            [s3_ref[c, pl.ds(j[c, 0], 1), :] for c in range(_C)], axis=0
        )
        pos = jnp.min(jnp.where(blocks == m, lane_i, 127), axis=1, keepdims=True)
        idx = jnp.minimum(j * 128 + pos, _N - 1)

        blocks_new = jnp.where(lane_i == pos, _NEG, blocks)
        for c in range(_C):
            s3_ref[c, pl.ds(j[c, 0], 1), :] = blocks_new[c : c + 1, :]
        bm = jnp.max(blocks_new, axis=1, keepdims=True)
        bmax = jnp.where(lane_b == j, bm, bmax)

        rows = [boxes_n4_ref[pl.ds(idx[c, 0], 1), :] for c in range(_C)]
        sel = jnp.concatenate(rows, axis=0)
        cy1 = sel[:, 0:1]
        cx1 = sel[:, 1:2]
        cy2 = sel[:, 2:3]
        cx2 = sel[:, 3:4]
        ca = jnp.maximum(cy2 - cy1, 0.0) * jnp.maximum(cx2 - cx1, 0.0)

        ka = jnp.maximum(ky2 - ky1, 0.0) * jnp.maximum(kx2 - kx1, 0.0)
        yy1 = jnp.maximum(ky1, cy1)
        xx1 = jnp.maximum(kx1, cx1)
        yy2 = jnp.minimum(ky2, cy2)
        xx2 = jnp.minimum(kx2, cx2)
        inter = jnp.maximum(yy2 - yy1, 0.0) * jnp.maximum(xx2 - xx1, 0.0)
        union = jnp.maximum(ka + ca - inter, 1e-9)
        iou = inter / union
        suppressed = jnp.any(iou > _IOU_T, axis=1, keepdims=True)

        accept = jnp.logical_and(keep, jnp.logical_not(suppressed))
        upd = jnp.logical_and(slot == cnt, accept)
        ky1 = jnp.where(upd, cy1, ky1)
        kx1 = jnp.where(upd, cx1, kx1)
        ky2 = jnp.where(upd, cy2, ky2)
        kx2 = jnp.where(upd, cx2, kx2)
        ks = jnp.where(upd, m, ks)
        cnt = cnt + accept.astype(jnp.int32)

        live_vec = jnp.logical_and(cnt < _M, keep).astype(jnp.int32)
        live = jnp.sum(live_vec, axis=0, keepdims=True)
        return (live[0, 0], cnt, bmax, ky1, kx1, ky2, kx2, ks)

    out = jax.lax.while_loop(cond, body, init)
    _, _, _, ky1, kx1, ky2, kx2, ks = out
    oy1_ref[...] = ky1
    ox1_ref[...] = kx1
    oy2_ref[...] = ky2
    ox2_ref[...] = kx2
    os_ref[...] = ks


def kernel(boxes, box_scores):
    scores_t = box_scores.T
    scores_p = jnp.pad(scores_t, ((0, 0), (0, _NP - _N))).reshape(_C, _NB, 128)
    oy1, ox1, oy2, ox2, osc = pl.pallas_call(
        _nms_lazy_kernel,
        out_shape=[jax.ShapeDtypeStruct((_C, _M), jnp.float32)] * 5,
        scratch_shapes=[pltpu.VMEM((_C, _NB, 128), jnp.float32)],
    )(scores_p, boxes)
    box_array = jnp.stack([oy1, ox1, oy2, ox2], axis=-1).reshape(-1, 4)
    score_array = osc.reshape(-1)
    class_array = jnp.repeat(jnp.arange(_C, dtype=jnp.int32), _M)
    return box_array, score_array, class_array
